# baseline (speedup 1.0000x reference)
import functools

import jax
import jax.numpy as jnp
from jax import lax
from jax.experimental import pallas as pl
from jax.experimental.pallas import tpu as pltpu
from jax.experimental.pallas import tpu_sc as plsc

B = 100000
D = 128
C = 400
NSUB = 5
SUB = C // NSUB
PIECES = ((0, 2), (2, 2), (4, 1))
NCHUNK = B // C
_info = plsc.get_sparse_core_info()
NC = _info.num_cores
NS = _info.num_subcores
NW = NC * NS
MAX_T = -(-NCHUNK // NW)
NFULL = NCHUNK - NW * (MAX_T - 1)

_mesh = plsc.VectorSubcoreMesh(core_axis_name="c", subcore_axis_name="s")


@functools.partial(
    pl.kernel,
    out_type=jax.ShapeDtypeStruct((B, D), jnp.float32),
    mesh=_mesh,
    scratch_types=[
        pltpu.VMEM((MAX_T * C,), jnp.int32),
        pltpu.VMEM((2, C, D), jnp.float32),
        pltpu.VMEM_SHARED((16, D), jnp.float32),
        pltpu.SemaphoreType.DMA,
        pltpu.SemaphoreType.DMA,
        pltpu.SemaphoreType.DMA,
    ],
)
def _emb_lookup(idx_hbm, table_hbm, out_hbm, idx_v, rows_v, table_sh,
                gsem, ssem0, ssem1):
    sid = lax.axis_index("s")
    wid = sid * NC + lax.axis_index("c")
    full = wid < NFULL
    start = jnp.where(full, wid * MAX_T,
                      NFULL * MAX_T + (wid - NFULL) * (MAX_T - 1))
    ssems = (ssem0, ssem1)

    cp_idx = pltpu.async_copy(idx_hbm.at[pl.ds(start * C, (MAX_T - 1) * C)],
                              idx_v.at[pl.ds(0, (MAX_T - 1) * C)], gsem)

    @pl.when(full)
    def _():
        pltpu.async_copy(idx_hbm.at[pl.ds((start + MAX_T - 1) * C, C)],
                         idx_v.at[pl.ds((MAX_T - 1) * C, C)], gsem)

    @pl.when(sid == 0)
    def _():
        pltpu.sync_copy(table_hbm, rows_v.at[0, pl.ds(0, 16)])
        pltpu.sync_copy(rows_v.at[0, pl.ds(0, 16)], table_sh)

    plsc.subcore_barrier()
    cp_idx.wait()

    @pl.when(full)
    def _():
        pltpu.make_async_copy(idx_hbm.at[pl.ds((start + MAX_T - 1) * C, C)],
                              idx_v.at[pl.ds((MAX_T - 1) * C, C)], gsem).wait()

    def piece_copy(b, t, p):
        j0, nj = PIECES[p]
        return pltpu.make_async_copy(
            rows_v.at[b, pl.ds(j0 * SUB, nj * SUB)],
            out_hbm.at[pl.ds((start + t) * C + j0 * SUB, nj * SUB)],
            ssems[b])

    def wait_scatter(b):
        for p in range(len(PIECES)):
            piece_copy(b, 0, p).wait()

    def do_chunk(t):
        b = t % 2
        if t >= 2:
            wait_scatter(b)
        copies = [
            pltpu.async_copy(
                table_sh.at[idx_v.at[pl.ds((t * NSUB + j) * SUB, SUB)]],
                rows_v.at[b, pl.ds(j * SUB, SUB)],
                gsem,
            )
            for j in range(NSUB)
        ]
        for p, (j0, nj) in enumerate(PIECES):
            for cp in copies[j0:j0 + nj]:
                cp.wait()
            piece_copy(b, t, p).start()

    for t in range(MAX_T - 1):
        do_chunk(t)

    @pl.when(full)
    def _():
        do_chunk(MAX_T - 1)

    wait_scatter(0)
    wait_scatter(1)


def kernel(node_type_indices, table):
    return _emb_lookup(node_type_indices.astype(jnp.int32), table)


# speedup vs baseline: 1.0050x; 1.0050x over previous
import functools

import jax
import jax.numpy as jnp
from jax import lax
from jax.experimental import pallas as pl
from jax.experimental.pallas import tpu as pltpu
from jax.experimental.pallas import tpu_sc as plsc

B = 100000
D = 128
C = 400
NSUB = 4
SUB = C // NSUB
PIECE = 2 * SUB
NCHUNK = B // C
_info = plsc.get_sparse_core_info()
NC = _info.num_cores
NS = _info.num_subcores
NW = NC * NS
MAX_T = -(-NCHUNK // NW)
NFULL = NCHUNK - NW * (MAX_T - 1)

_mesh = plsc.VectorSubcoreMesh(core_axis_name="c", subcore_axis_name="s")


@functools.partial(
    pl.kernel,
    out_type=jax.ShapeDtypeStruct((B, D), jnp.float32),
    mesh=_mesh,
    scratch_types=[
        pltpu.VMEM((MAX_T, NSUB, SUB), jnp.int32),
        pltpu.VMEM((2, C, D), jnp.float32),
        pltpu.VMEM_SHARED((16, D), jnp.float32),
        pltpu.SemaphoreType.DMA,
        pltpu.SemaphoreType.DMA,
        pltpu.SemaphoreType.DMA,
    ],
)
def _emb_lookup(idx_hbm, table_hbm, out_hbm, idx_v, rows_v, table_sh,
                gsem, ssem0, ssem1):
    sid = lax.axis_index("s")
    wid = sid * NC + lax.axis_index("c")
    full = wid < NFULL
    start = jnp.where(full, wid * MAX_T,
                      NFULL * MAX_T + (wid - NFULL) * (MAX_T - 1))
    ssems = (ssem0, ssem1)

    cp_idx = pltpu.async_copy(idx_hbm.at[pl.ds(start, MAX_T)], idx_v, gsem)

    @pl.when(sid == 0)
    def _():
        pltpu.sync_copy(table_hbm, rows_v.at[0, pl.ds(0, 16)])
        pltpu.sync_copy(rows_v.at[0, pl.ds(0, 16)], table_sh)

    plsc.subcore_barrier()
    cp_idx.wait()

    def wait_scatter(b):
        for p in range(C // PIECE):
            pltpu.make_async_copy(rows_v.at[b, pl.ds(p * PIECE, PIECE)],
                                  out_hbm.at[pl.ds(p * PIECE, PIECE)],
                                  ssems[b]).wait()

    def do_chunk(t):
        b = t % 2
        if t >= 2:
            wait_scatter(b)
        copies = [
            pltpu.async_copy(
                table_sh.at[idx_v.at[t, j]],
                rows_v.at[b, pl.ds(j * SUB, SUB)],
                gsem,
            )
            for j in range(NSUB)
        ]
        per_piece = PIECE // SUB
        for p in range(C // PIECE):
            for cp in copies[p * per_piece:(p + 1) * per_piece]:
                cp.wait()
            pltpu.async_copy(
                rows_v.at[b, pl.ds(p * PIECE, PIECE)],
                out_hbm.at[pl.ds((start + t) * C + p * PIECE, PIECE)],
                ssems[b])

    for t in range(MAX_T - 1):
        do_chunk(t)

    @pl.when(full)
    def _():
        do_chunk(MAX_T - 1)

    wait_scatter(0)
    wait_scatter(1)


def kernel(node_type_indices, table):
    idx = node_type_indices.astype(jnp.int32).reshape(NCHUNK, C)
    idx = jnp.concatenate([idx, jnp.zeros((1, C), jnp.int32)], axis=0)
    idx = idx.reshape(NCHUNK + 1, NSUB, SUB)
    return _emb_lookup(idx, table)
